# baseline (speedup 1.0000x reference)
import dataclasses
import functools

import jax
import jax.numpy as jnp
from jax import lax
from jax.experimental import pallas as pl
from jax.experimental.pallas import tpu as pltpu
from jax.experimental.pallas import tpu_sc as plsc

F0, F1 = 3, 3
_NUM_SC_CORES = 2
_NUM_SC_SUBCORES = 16
_LANES = 16


def _make_sc_scatter(npad, tot, chunk):
    nw = _NUM_SC_CORES * _NUM_SC_SUBCORES
    assert tot % nw == 0 and chunk % _LANES == 0 and npad % _LANES == 0

    mesh = plsc.VectorSubcoreMesh(core_axis_name="c", subcore_axis_name="s")
    cp = pltpu.CompilerParams()
    if "needs_layout_passes" in pltpu.CompilerParams.__dataclass_fields__:
        cp = dataclasses.replace(cp, needs_layout_passes=False)

    @functools.partial(
        pl.kernel,
        mesh=mesh,
        compiler_params=cp,
        out_type=jax.ShapeDtypeStruct((tot,), jnp.float32),
        scratch_types=[
            pltpu.VMEM((npad,), jnp.int32),
            pltpu.VMEM((npad,), jnp.float32),
            pltpu.VMEM((chunk,), jnp.float32),
        ],
    )
    def sc_scatter(flat_hbm, vals_hbm, out_hbm, idx_v, vals_v, chunk_v):
        cid = lax.axis_index("c")
        sid = lax.axis_index("s")
        wid = sid * _NUM_SC_CORES + cid
        base = pl.multiple_of(wid * chunk, 8)

        pltpu.sync_copy(flat_hbm, idx_v)
        pltpu.sync_copy(vals_hbm, vals_v)

        zero = jnp.zeros((_LANES,), jnp.float32)

        @pl.loop(0, chunk, step=_LANES)
        def _(i):
            chunk_v[pl.ds(i, _LANES)] = zero

        @pl.loop(0, npad, step=_LANES)
        def _(i):
            flat = idx_v[pl.ds(i, _LANES)]
            v = vals_v[pl.ds(i, _LANES)]
            loc = flat - base
            m = (loc >= 0) & (loc < chunk)
            loc = jnp.where(m, loc, 0)
            plsc.store_scatter(chunk_v, [loc], v, mask=m)

        pltpu.sync_copy(chunk_v, out_hbm.at[pl.ds(base, chunk)])

    return sc_scatter


_ROWS_PER_STEP = 6


def _make_tc_body(hout, wout, in_f, nf, rps):
    def body(x_ref, w_ref, b_ref, o_ref):
        i0 = pl.program_id(1) * rps
        for r in range(rps):
            i = i0 + r
            parts = []
            for di in range(F0):
                for dj in range(F1):
                    parts.append(x_ref[0, i + di, pl.ds(dj, wout), :])
            p = jnp.concatenate(parts, axis=-1)
            acc = jnp.dot(p, w_ref[...], preferred_element_type=jnp.float32)
            o_ref[0, r] = acc + b_ref[...]

    return body


def _tc_conv(x, wd, bias2d):
    b, h, w, c = x.shape
    hout, wout = h - F0 + 1, w - F1 + 1
    in_f, nf = wd.shape
    rps = _ROWS_PER_STEP
    assert hout % rps == 0
    return pl.pallas_call(
        _make_tc_body(hout, wout, in_f, nf, rps),
        grid=(b, hout // rps),
        in_specs=[
            pl.BlockSpec((1, h, w, c), lambda bb, ii: (bb, 0, 0, 0)),
            pl.BlockSpec((in_f, nf), lambda bb, ii: (0, 0)),
            pl.BlockSpec((1, nf), lambda bb, ii: (0, 0)),
        ],
        out_specs=pl.BlockSpec((1, rps, wout, nf), lambda bb, ii: (bb, ii, 0, 0)),
        out_shape=jax.ShapeDtypeStruct((b, hout, wout, nf), jnp.float32),
    )(x, wd, bias2d)


def kernel(inputs, kernel_vals, bias, row_idx, col_idx):
    b, h, w, c = inputs.shape
    nf = bias.shape[0]
    nnz = kernel_vals.shape[0]

    cp_ = 128
    in_fp = cp_ * F0 * F1
    x_pad = jnp.pad(inputs, ((0, 0), (0, 0), (0, 0), (0, cp_ - c))).astype(
        jnp.bfloat16
    )

    row32 = row_idx.astype(jnp.int32)
    col32 = col_idx.astype(jnp.int32)
    row_pad = (row32 // c) * cp_ + row32 % c
    flat = row_pad * nf + col32

    tot = in_fp * nf
    chunk = tot // (_NUM_SC_CORES * _NUM_SC_SUBCORES)
    npad = ((nnz + _LANES - 1) // _LANES) * _LANES
    pad = npad - nnz
    flat = jnp.concatenate([flat, jnp.full((pad,), tot, jnp.int32)])
    vals = jnp.concatenate([kernel_vals, jnp.zeros((pad,), jnp.float32)])

    w_flat = _make_sc_scatter(npad, tot, chunk)(flat, vals)
    wd = w_flat.reshape(in_fp, nf).astype(jnp.bfloat16)

    return _tc_conv(x_pad, wd, bias.reshape(1, nf))


# speedup vs baseline: 1.0629x; 1.0629x over previous
import dataclasses
import functools

import jax
import jax.numpy as jnp
from jax import lax
from jax.experimental import pallas as pl
from jax.experimental.pallas import tpu as pltpu
from jax.experimental.pallas import tpu_sc as plsc

F0, F1 = 3, 3
_NUM_SC_CORES = 2
_NUM_SC_SUBCORES = 16
_LANES = 16


def _make_sc_scatter(npad, tot, chunk):
    nw = _NUM_SC_CORES * _NUM_SC_SUBCORES
    assert tot % nw == 0 and chunk % _LANES == 0 and npad % _LANES == 0

    mesh = plsc.VectorSubcoreMesh(core_axis_name="c", subcore_axis_name="s")
    cp = pltpu.CompilerParams()
    if "needs_layout_passes" in pltpu.CompilerParams.__dataclass_fields__:
        cp = dataclasses.replace(cp, needs_layout_passes=False)

    @functools.partial(
        pl.kernel,
        mesh=mesh,
        compiler_params=cp,
        out_type=jax.ShapeDtypeStruct((tot,), jnp.float32),
        scratch_types=[
            pltpu.VMEM((npad,), jnp.int32),
            pltpu.VMEM((npad,), jnp.float32),
            pltpu.VMEM((chunk,), jnp.float32),
        ],
    )
    def sc_scatter(flat_hbm, vals_hbm, out_hbm, idx_v, vals_v, chunk_v):
        cid = lax.axis_index("c")
        sid = lax.axis_index("s")
        wid = sid * _NUM_SC_CORES + cid
        base = pl.multiple_of(wid * chunk, 8)

        pltpu.sync_copy(flat_hbm, idx_v)
        pltpu.sync_copy(vals_hbm, vals_v)

        zero = jnp.zeros((_LANES,), jnp.float32)

        @pl.loop(0, chunk, step=_LANES)
        def _(i):
            chunk_v[pl.ds(i, _LANES)] = zero

        @pl.loop(0, npad, step=_LANES)
        def _(i):
            flat = idx_v[pl.ds(i, _LANES)]
            v = vals_v[pl.ds(i, _LANES)]
            loc = flat - base
            m = (loc >= 0) & (loc < chunk)
            loc = jnp.where(m, loc, 0)
            plsc.store_scatter(chunk_v, [loc], v, mask=m)

        pltpu.sync_copy(chunk_v, out_hbm.at[pl.ds(base, chunk)])

    return sc_scatter


_ROWS_PER_STEP = 37


def _make_tc_body(hout, wout, in_f, nf, rps):
    def body(x_ref, w_ref, b_ref, o_ref):
        i0 = pl.program_id(1) * rps
        for r in range(rps):
            i = i0 + r
            parts = []
            for di in range(F0):
                for dj in range(F1):
                    parts.append(x_ref[0, i + di, pl.ds(dj, wout), :])
            p = jnp.concatenate(parts, axis=-1)
            acc = jnp.dot(p, w_ref[...], preferred_element_type=jnp.float32)
            o_ref[0, r] = acc + b_ref[...]

    return body


def _tc_conv(x, wd, bias2d):
    b, h, w, c = x.shape
    hout, wout = h - F0 + 1, w - F1 + 1
    in_f, nf = wd.shape
    rps = _ROWS_PER_STEP
    assert hout % rps == 0
    return pl.pallas_call(
        _make_tc_body(hout, wout, in_f, nf, rps),
        grid=(b, hout // rps),
        in_specs=[
            pl.BlockSpec((1, h, w, c), lambda bb, ii: (bb, 0, 0, 0)),
            pl.BlockSpec((in_f, nf), lambda bb, ii: (0, 0)),
            pl.BlockSpec((1, nf), lambda bb, ii: (0, 0)),
        ],
        out_specs=pl.BlockSpec((1, rps, wout, nf), lambda bb, ii: (bb, ii, 0, 0)),
        out_shape=jax.ShapeDtypeStruct((b, hout, wout, nf), jnp.float32),
    )(x, wd, bias2d)


def kernel(inputs, kernel_vals, bias, row_idx, col_idx):
    b, h, w, c = inputs.shape
    nf = bias.shape[0]
    nnz = kernel_vals.shape[0]

    cp_ = 128
    in_fp = cp_ * F0 * F1
    x_pad = jnp.pad(inputs, ((0, 0), (0, 0), (0, 0), (0, cp_ - c))).astype(
        jnp.bfloat16
    )

    row32 = row_idx.astype(jnp.int32)
    col32 = col_idx.astype(jnp.int32)
    row_pad = (row32 // c) * cp_ + row32 % c
    flat = row_pad * nf + col32

    tot = in_fp * nf
    chunk = tot // (_NUM_SC_CORES * _NUM_SC_SUBCORES)
    npad = ((nnz + _LANES - 1) // _LANES) * _LANES
    pad = npad - nnz
    flat = jnp.concatenate([flat, jnp.full((pad,), tot, jnp.int32)])
    vals = jnp.concatenate([kernel_vals, jnp.zeros((pad,), jnp.float32)])

    w_flat = _make_sc_scatter(npad, tot, chunk)(flat, vals)
    wd = w_flat.reshape(in_fp, nf).astype(jnp.bfloat16)

    return _tc_conv(x_pad, wd, bias.reshape(1, nf))


# speedup vs baseline: 1.9357x; 1.8210x over previous
import dataclasses
import functools

import jax
import jax.numpy as jnp
from jax import lax
from jax.experimental import pallas as pl
from jax.experimental.pallas import tpu as pltpu
from jax.experimental.pallas import tpu_sc as plsc

F0, F1 = 3, 3
_NUM_SC_CORES = 2
_NUM_SC_SUBCORES = 16
_LANES = 16

_ROWS_PER_STEP = 6


def _make_sc_scatter(npad, tot, chunk):
    nw = _NUM_SC_CORES * _NUM_SC_SUBCORES
    assert tot % nw == 0 and chunk % _LANES == 0 and npad % _LANES == 0

    mesh = plsc.VectorSubcoreMesh(core_axis_name="c", subcore_axis_name="s")
    cp = pltpu.CompilerParams()
    if "needs_layout_passes" in pltpu.CompilerParams.__dataclass_fields__:
        cp = dataclasses.replace(cp, needs_layout_passes=False)

    @functools.partial(
        pl.kernel,
        mesh=mesh,
        compiler_params=cp,
        out_type=jax.ShapeDtypeStruct((tot,), jnp.float32),
        scratch_types=[
            pltpu.VMEM((npad,), jnp.int32),
            pltpu.VMEM((npad,), jnp.float32),
            pltpu.VMEM((chunk,), jnp.float32),
        ],
    )
    def sc_scatter(flat_hbm, vals_hbm, out_hbm, idx_v, vals_v, chunk_v):
        cid = lax.axis_index("c")
        sid = lax.axis_index("s")
        wid = sid * _NUM_SC_CORES + cid
        base = pl.multiple_of(wid * chunk, 8)

        pltpu.sync_copy(flat_hbm, idx_v)
        pltpu.sync_copy(vals_hbm, vals_v)

        zero = jnp.zeros((_LANES,), jnp.float32)

        @pl.loop(0, chunk, step=_LANES)
        def _(i):
            chunk_v[pl.ds(i, _LANES)] = zero

        @pl.loop(0, npad, step=_LANES)
        def _(i):
            flat = idx_v[pl.ds(i, _LANES)]
            v = vals_v[pl.ds(i, _LANES)]
            loc = flat - base
            m = (loc >= 0) & (loc < chunk)
            loc = jnp.where(m, loc, 0)
            plsc.store_scatter(chunk_v, [loc], v, mask=m)

        pltpu.sync_copy(chunk_v, out_hbm.at[pl.ds(base, chunk)])

    return sc_scatter


def _make_tc_body(wtot, wout, cin, nf, rps):
    def body(x_ref, w_ref, b_ref, o_ref):
        i0 = pl.program_id(1) * rps
        for r in range(rps):
            i = i0 + r
            xs = jnp.concatenate(
                [x_ref[0, i + di, :, :] for di in range(F0)], axis=0
            ).astype(jnp.bfloat16)
            accs = [
                jnp.dot(w_ref[dj], xs, preferred_element_type=jnp.float32)
                for dj in range(F1)
            ]
            out = (
                accs[0][:, 0:wout]
                + accs[1][:, 1 : wout + 1]
                + accs[2][:, 2 : wout + 2]
            )
            o_ref[0, r] = out + b_ref[...]

    return body


def _tc_conv(x_t, wd, bias2d):
    b, h, cin, wtot = x_t.shape
    hout, wout = h - F0 + 1, wtot - F1 + 1
    nf = wd.shape[1]
    rps = _ROWS_PER_STEP
    assert hout % rps == 0
    return pl.pallas_call(
        _make_tc_body(wtot, wout, cin, nf, rps),
        grid=(b, hout // rps),
        in_specs=[
            pl.BlockSpec((1, h, cin, wtot), lambda bb, ii: (bb, 0, 0, 0)),
            pl.BlockSpec((F1, nf, F0 * cin), lambda bb, ii: (0, 0, 0)),
            pl.BlockSpec((nf, 1), lambda bb, ii: (0, 0)),
        ],
        out_specs=pl.BlockSpec((1, rps, nf, wout), lambda bb, ii: (bb, ii, 0, 0)),
        out_shape=jax.ShapeDtypeStruct((b, hout, nf, wout), jnp.float32),
    )(x_t, wd, bias2d)


def kernel(inputs, kernel_vals, bias, row_idx, col_idx):
    b, h, w, c = inputs.shape
    nf = bias.shape[0]
    nnz = kernel_vals.shape[0]

    x_t = jnp.transpose(inputs, (0, 1, 3, 2))

    row32 = row_idx.astype(jnp.int32)
    col32 = col_idx.astype(jnp.int32)
    blk = row32 // c
    ch = row32 % c
    di = blk // F1
    dj = blk % F1
    flat = (dj * nf + col32) * (F0 * c) + di * c + ch

    tot = F1 * nf * F0 * c
    chunk = tot // (_NUM_SC_CORES * _NUM_SC_SUBCORES)
    npad = ((nnz + _LANES - 1) // _LANES) * _LANES
    pad = npad - nnz
    flat = jnp.concatenate([flat, jnp.full((pad,), tot, jnp.int32)])
    vals = jnp.concatenate([kernel_vals, jnp.zeros((pad,), jnp.float32)])

    w_flat = _make_sc_scatter(npad, tot, chunk)(flat, vals)
    wd = w_flat.reshape(F1, nf, F0 * c).astype(jnp.bfloat16)

    out_t = _tc_conv(x_t, wd, bias.reshape(nf, 1))
    return jnp.transpose(out_t, (0, 1, 3, 2))
